# baseline (speedup 1.0000x reference)
import functools

import jax
import jax.numpy as jnp
from jax import lax
from jax.experimental import pallas as pl
from jax.experimental.pallas import tpu as pltpu
from jax.experimental.pallas import tpu_sc as plsc

_PNUM = 128
_BATCH = 1024
_NCHUNK = _PNUM // 16


def _smooth_l1_sum(p, g, acc):
    d = p - g
    ad = jnp.abs(d)
    m = jnp.minimum(ad, 1.0)
    return acc + m * (ad - 0.5 * m)


def _make_sc_kernel(n_workers, b_per_w):
    mesh = plsc.VectorSubcoreMesh(core_axis_name="c", subcore_axis_name="s")

    @functools.partial(
        pl.kernel,
        mesh=mesh,
        out_type=jax.ShapeDtypeStruct((n_workers, 16), jnp.float32),
        scratch_types=[
            pltpu.VMEM((b_per_w * _PNUM,), jnp.float32),
            pltpu.VMEM((b_per_w * _PNUM,), jnp.float32),
            pltpu.VMEM((b_per_w * 2 * _PNUM,), jnp.float32),
            pltpu.VMEM((b_per_w * 2 * _PNUM,), jnp.float32),
            pltpu.VMEM((16,), jnp.float32),
        ],
        compiler_params=pltpu.CompilerParams(needs_layout_passes=False),
    )
    def sc_kernel(px_hbm, py_hbm, gx_hbm, gy_hbm, out_hbm,
                  px_v, py_v, gx_v, gy_v, out_v):
        nc = 2
        wid = lax.axis_index("s") * nc + lax.axis_index("c")
        base = wid * b_per_w
        pltpu.sync_copy(px_hbm.at[pl.ds(base * _PNUM, b_per_w * _PNUM)], px_v)
        pltpu.sync_copy(py_hbm.at[pl.ds(base * _PNUM, b_per_w * _PNUM)], py_v)
        pltpu.sync_copy(
            gx_hbm.at[pl.ds(base * 2 * _PNUM, b_per_w * 2 * _PNUM)], gx_v)
        pltpu.sync_copy(
            gy_hbm.at[pl.ds(base * 2 * _PNUM, b_per_w * 2 * _PNUM)], gy_v)

        lane = jnp.arange(16, dtype=jnp.int32)
        zero16 = jnp.zeros((16,), jnp.int32)

        def batch_body(b, bacc):
            gbase = b * 2 * _PNUM
            pbase = b * _PNUM

            init = tuple(
                jnp.zeros((16,), jnp.float32) for _ in range(_NCHUNK))

            @plsc.parallel_loop(0, _PNUM, carry=init)
            def accs(j, accs):
                sidx = zero16 + (pbase + j)
                px_s = plsc.load_gather(px_v, [sidx])
                py_s = plsc.load_gather(py_v, [sidx])
                idx0 = gbase + j + lane
                out = []
                for g in range(_NCHUNK):
                    idx = idx0 + g * 16
                    gx = plsc.load_gather(gx_v, [idx])
                    gy = plsc.load_gather(gy_v, [idx])
                    acc = _smooth_l1_sum(px_s, gx, accs[g])
                    acc = _smooth_l1_sum(py_s, gy, acc)
                    out.append(acc)
                return tuple(out)

            m = accs[0]
            for g in range(1, _NCHUNK):
                m = jnp.minimum(m, accs[g])
            return bacc + jnp.min(m)

        bacc = lax.fori_loop(0, b_per_w, batch_body, jnp.float32(0.0))
        out_v[...] = jnp.zeros((16,), jnp.float32) + bacc
        pltpu.sync_copy(out_v, out_hbm.at[wid])

    return sc_kernel


def _sl1(d):
    ad = jnp.abs(d)
    m = jnp.minimum(ad, 1.0)
    return m * (ad - 0.5 * m)


def _tc_body(px_ref, py_ref, gx_ref, gy_ref, out_ref):
    px = px_ref[...]
    py = py_ref[...]
    gx = gx_ref[...]
    gy = gy_ref[...]
    d = _sl1(px[:, :, None] - gx[:, None, :])
    d = d + _sl1(py[:, :, None] - gy[:, None, :])
    e = pltpu.roll(d, 0, axis=2, stride=1, stride_axis=1)
    dis = jnp.sum(e, axis=1)
    out_ref[...] = jnp.min(dis, axis=1, keepdims=True)


def _tc_mins(px, py, gx, gy, n_batch, tile):
    grid = n_batch // tile
    return pl.pallas_call(
        _tc_body,
        grid=(grid,),
        in_specs=[
            pl.BlockSpec((tile, _PNUM), lambda t: (t, 0)),
            pl.BlockSpec((tile, _PNUM), lambda t: (t, 0)),
            pl.BlockSpec((tile, _PNUM), lambda t: (t, 0)),
            pl.BlockSpec((tile, _PNUM), lambda t: (t, 0)),
        ],
        out_specs=pl.BlockSpec((tile, 1), lambda t: (t, 0)),
        out_shape=jax.ShapeDtypeStruct((n_batch, 1), jnp.float32),
    )(px, py, gx, gy)


@jax.jit
def kernel(pred, gt):
    px = pred[:, :, 0]
    py = pred[:, :, 1]
    ridx = (-jnp.arange(_PNUM)) % _PNUM
    gtr = gt[:, ridx, :]
    gx = gtr[:, :, 0]
    gy = gtr[:, :, 1]
    mins = _tc_mins(px, py, gx, gy, _BATCH, 128)
    return jnp.sum(mins) * (1.0 / (_BATCH * _PNUM))
